# Initial kernel scaffold; baseline (speedup 1.0000x reference)
import jax
import jax.numpy as jnp
from jax.experimental import pallas as pl


def kernel(node_ids, edge_index, W, M, emb):
    raise NotImplementedError("write your pallas kernel here")


# speedup vs baseline: 2.1420x; 2.1420x over previous
import functools

import jax
import jax.numpy as jnp
from jax import lax
from jax.experimental import pallas as pl
from jax.experimental.pallas import tpu as pltpu
from jax.experimental.pallas import tpu_sc as plsc

N = 10000
D = 128
E = 320000

NC = 2
NS = 16
NW = NC * NS
L = 16

N_PAD = NW * 384
N_G = 10240
WIN = N_G // NC
TRASH = 128
E_PAD = NW * 10240
NODE_CHUNKS = 3
EDGE_OUTER = 20
SUB = 8
WAVE = 4
ROWS_PW = N_PAD // NW
EROWS_PS = E_PAD // 128 // NS
GROWS_PS = (WIN + TRASH) // NS
WROWS_PS = WIN // NS

_MESH = plsc.VectorSubcoreMesh(
    core_axis_name="c", subcore_axis_name="s", num_cores=NC, num_subcores=NS)


def _sc_gather_e(emb, nids_1d):
  @functools.partial(
      pl.kernel,
      out_type=jax.ShapeDtypeStruct((N_PAD, D), jnp.float32),
      mesh=_MESH,
      scratch_types=[
          pltpu.VMEM((ROWS_PW,), jnp.int32),
          pltpu.VMEM((ROWS_PW, D), jnp.float32),
          pltpu.SemaphoreType.DMA,
      ],
  )
  def k(emb_hbm, nids_hbm, e_hbm, idx_v, rows_v, sem):
    cid = lax.axis_index("c")
    sid = lax.axis_index("s")
    wid = cid * NS + sid
    pltpu.sync_copy(nids_hbm.at[pl.ds(wid * ROWS_PW, ROWS_PW)], idx_v)
    descs = [
        pltpu.async_copy(emb_hbm.at[idx_v.at[pl.ds(c * 128, 128)]],
                         rows_v.at[pl.ds(c * 128, 128)], sem)
        for c in range(NODE_CHUNKS)
    ]
    for d in descs:
      d.wait()
    pltpu.sync_copy(rows_v, e_hbm.at[pl.ds(wid * ROWS_PW, ROWS_PW)])

  return k(emb, nids_1d)


def _sc_edge_agg(e, src_2d, dst_2d):
  @functools.partial(
      pl.kernel,
      out_type=jax.ShapeDtypeStruct((N_G, D), jnp.float32),
      mesh=_MESH,
      scratch_types=[
          pltpu.VMEM((SUB, 128), jnp.int32),
          pltpu.VMEM((SUB, 128), jnp.int32),
          pltpu.VMEM((WAVE * 128, D), jnp.float32),
          pltpu.VMEM_SHARED((WIN + TRASH, D), jnp.float32),
          pltpu.SemaphoreType.DMA,
      ],
  )
  def k(e_hbm, src_hbm, dst_hbm, g_hbm, src_v, dst_v, rows_v, g_sh, sem):
    cid = lax.axis_index("c")
    sid = lax.axis_index("s")
    base = cid * WIN

    zero16 = jnp.zeros((L,), jnp.float32)

    def zrow(i, carry):
      for j in range(D // L):
        rows_v[i, pl.ds(j * L, L)] = zero16
      return carry

    lax.fori_loop(0, WAVE * 128, zrow, None)
    pltpu.sync_copy(rows_v.at[pl.ds(0, GROWS_PS)],
                    g_sh.at[pl.ds(sid * GROWS_PS, GROWS_PS)])
    plsc.subcore_barrier()

    def edge_chunk(oc, carry):
      row0 = sid * EROWS_PS + oc * SUB
      pltpu.sync_copy(src_hbm.at[pl.ds(row0, SUB)], src_v)
      pltpu.sync_copy(dst_hbm.at[pl.ds(row0, SUB)], dst_v)
      for r in range(SUB):
        for j in range(128 // L):
          d = dst_v[r, pl.ds(j * L, L)]
          in_win = (d >= base) & (d < base + WIN)
          dst_v[r, pl.ds(j * L, L)] = jnp.where(
              in_win, d - base, WIN + (d & (TRASH - 1)))
      for w in range(SUB // WAVE):
        gdescs = [
            pltpu.async_copy(e_hbm.at[src_v.at[w * WAVE + r]],
                             rows_v.at[pl.ds(r * 128, 128)], sem)
            for r in range(WAVE)
        ]
        for d in gdescs:
          d.wait()
        for r in range(WAVE):
          pltpu.sync_copy(rows_v.at[pl.ds(r * 128, 128)],
                          g_sh.at[dst_v.at[w * WAVE + r]], add=True)
      return carry

    lax.fori_loop(0, EDGE_OUTER, edge_chunk, None)
    plsc.subcore_barrier()

    pltpu.sync_copy(g_sh.at[pl.ds(sid * WROWS_PS, WROWS_PS)],
                    g_hbm.at[pl.ds(base + sid * WROWS_PS, WROWS_PS)])

  return k(e, src_2d, dst_2d)


TB = 512
TC_GRID = N_G // TB


def _tc_body(e_ref, g_ref, w_ref, m_ref, o_ref, acc_ref):
  b = pl.program_id(0)
  x = e_ref[...]
  s = lax.dot_general(x, w_ref[...], (((1,), (1,)), ((), ())),
                      preferred_element_type=jnp.float32)
  s += lax.dot_general(g_ref[...], m_ref[...], (((1,), (1,)), ((), ())),
                       preferred_element_type=jnp.float32)
  s = jnp.maximum(s, 0.0)
  row = b * TB + lax.broadcasted_iota(jnp.int32, (TB, 1), 0)
  s = jnp.where(row < N, s, 0.0)

  @pl.when(b == 0)
  def _():
    acc_ref[...] = jnp.zeros((8, D), jnp.float32)

  acc_ref[...] += jnp.sum(s.reshape(TB // 8, 8, D), axis=0)

  @pl.when(b == TC_GRID - 1)
  def _():
    r = jnp.sum(acc_ref[...], axis=0, keepdims=True)
    mx = jnp.max(r)
    ex = jnp.exp(r - mx)
    o_ref[...] = ex / jnp.sum(ex)


def _tc_reduce(e, g, W, M):
  return pl.pallas_call(
      _tc_body,
      grid=(TC_GRID,),
      in_specs=[
          pl.BlockSpec((TB, D), lambda b: (b, 0)),
          pl.BlockSpec((TB, D), lambda b: (b, 0)),
          pl.BlockSpec((D, D), lambda b: (0, 0)),
          pl.BlockSpec((D, D), lambda b: (0, 0)),
      ],
      out_specs=pl.BlockSpec((1, D), lambda b: (0, 0)),
      out_shape=jax.ShapeDtypeStruct((1, D), jnp.float32),
      scratch_shapes=[pltpu.VMEM((8, D), jnp.float32)],
  )(e, g, W, M)


@jax.jit
def kernel(node_ids, edge_index, W, M, emb):
  nids = jnp.concatenate(
      [node_ids, jnp.zeros((N_PAD - N,), jnp.int32)]).astype(jnp.int32)
  src = jnp.concatenate(
      [edge_index[0], jnp.zeros((E_PAD - E,), jnp.int32)]).astype(jnp.int32)
  dst = jnp.concatenate(
      [edge_index[1], jnp.full((E_PAD - E,), N, jnp.int32)]).astype(jnp.int32)

  e = _sc_gather_e(emb, nids)
  g = _sc_edge_agg(e, src.reshape(E_PAD // 128, 128),
                   dst.reshape(E_PAD // 128, 128))
  out = _tc_reduce(e, g, W, M)
  return out.reshape(D, 1)


# speedup vs baseline: 2.3094x; 1.0782x over previous
import functools

import jax
import jax.numpy as jnp
from jax import lax
from jax.experimental import pallas as pl
from jax.experimental.pallas import tpu as pltpu
from jax.experimental.pallas import tpu_sc as plsc

N = 10000
D = 128
E = 320000

NC = 2
NS = 16
NW = NC * NS
L = 16

N_PAD = NW * 384
N_G = 10240
WIN = N_G // NC
TRASH = 128
E_PAD = NW * 10240
NODE_CHUNKS = 3
RING = 3
CH = 32
ROWS_PW = N_PAD // NW
EROWS_PS = E_PAD // 128 // NS
NCH = EROWS_PS // CH
GROWS_PS = (WIN + TRASH) // NS
WROWS_PS = WIN // NS

_MESH = plsc.VectorSubcoreMesh(
    core_axis_name="c", subcore_axis_name="s", num_cores=NC, num_subcores=NS)


def _sc_gather_e(emb, nids_1d):
  @functools.partial(
      pl.kernel,
      out_type=jax.ShapeDtypeStruct((N_PAD, D), jnp.float32),
      mesh=_MESH,
      scratch_types=[
          pltpu.VMEM((ROWS_PW,), jnp.int32),
          pltpu.VMEM((ROWS_PW, D), jnp.float32),
          pltpu.SemaphoreType.DMA,
      ],
  )
  def k(emb_hbm, nids_hbm, e_hbm, idx_v, rows_v, sem):
    cid = lax.axis_index("c")
    sid = lax.axis_index("s")
    wid = cid * NS + sid
    pltpu.sync_copy(nids_hbm.at[pl.ds(wid * ROWS_PW, ROWS_PW)], idx_v)
    descs = [
        pltpu.async_copy(emb_hbm.at[idx_v.at[pl.ds(c * 128, 128)]],
                         rows_v.at[pl.ds(c * 128, 128)], sem)
        for c in range(NODE_CHUNKS)
    ]
    for d in descs:
      d.wait()
    pltpu.sync_copy(rows_v, e_hbm.at[pl.ds(wid * ROWS_PW, ROWS_PW)])

  return k(emb, nids_1d)


def _sc_edge_agg(e, src_2d, dst_2d):
  @functools.partial(
      pl.kernel,
      out_type=jax.ShapeDtypeStruct((N_G, D), jnp.float32),
      mesh=_MESH,
      scratch_types=[
          pltpu.VMEM((2 * CH, 128), jnp.int32),
          pltpu.VMEM((2 * CH, 128), jnp.int32),
          pltpu.VMEM((RING * 128, D), jnp.float32),
          pltpu.VMEM_SHARED((WIN + TRASH, D), jnp.float32),
          pltpu.SemaphoreType.DMA,
          pltpu.SemaphoreType.DMA,
          pltpu.SemaphoreType.DMA,
      ],
  )
  def k(e_hbm, src_hbm, dst_hbm, g_hbm, src_v, dst_v, rows_v, g_sh,
        sem_g, sem_s, sem_i):
    cid = lax.axis_index("c")
    sid = lax.axis_index("s")
    base = cid * WIN

    zero16 = jnp.zeros((L,), jnp.float32)

    def zrow(i, carry):
      for j in range(D // L):
        rows_v[i, pl.ds(j * L, L)] = zero16
      return carry

    lax.fori_loop(0, 128, zrow, None)
    for off, nr in ((0, 128), (128, 128), (256, 72)):
      pltpu.sync_copy(rows_v.at[pl.ds(0, nr)],
                      g_sh.at[pl.ds(sid * GROWS_PS + off, nr)])
    plsc.subcore_barrier()

    erow0 = sid * EROWS_PS
    pltpu.sync_copy(src_hbm.at[pl.ds(erow0, CH)], src_v.at[pl.ds(0, CH)])
    pltpu.sync_copy(dst_hbm.at[pl.ds(erow0, CH)], dst_v.at[pl.ds(0, CH)])

    def chunk_body(c, carry):
      half = lax.rem(c, 2) * CH
      nxt = lax.rem(c + 1, 2) * CH

      @pl.when(c + 1 < NCH)
      def _():
        pltpu.async_copy(src_hbm.at[pl.ds(erow0 + (c + 1) * CH, CH)],
                         src_v.at[pl.ds(nxt, CH)], sem_i)
        pltpu.async_copy(dst_hbm.at[pl.ds(erow0 + (c + 1) * CH, CH)],
                         dst_v.at[pl.ds(nxt, CH)], sem_i)

      for i in range(RING):
        pltpu.async_copy(e_hbm.at[src_v.at[half + i]],
                         rows_v.at[pl.ds(i * 128, 128)], sem_g)

      def step(i, carry2):
        slot = lax.rem(i, RING)
        hi = half + i
        pltpu.make_async_copy(
            e_hbm.at[pl.ds(0, 128)],
            rows_v.at[pl.ds(slot * 128, 128)], sem_g).wait()
        for j in range(128 // L):
          d = dst_v[hi, pl.ds(j * L, L)]
          in_win = (d >= base) & (d < base + WIN)
          dst_v[hi, pl.ds(j * L, L)] = jnp.where(
              in_win, d - base, WIN + (d & (TRASH - 1)))
        pltpu.async_copy(rows_v.at[pl.ds(slot * 128, 128)],
                         g_sh.at[dst_v.at[hi]], sem_s, add=True)

        @pl.when(i + RING < CH)
        def _():
          pltpu.make_async_copy(
              e_hbm.at[pl.ds(0, 128)],
              rows_v.at[pl.ds(slot * 128, 128)], sem_s).wait()
          pltpu.async_copy(e_hbm.at[src_v.at[half + i + RING]],
                           rows_v.at[pl.ds(slot * 128, 128)], sem_g)
        return carry2

      lax.fori_loop(0, CH, step, None)
      for i in range(RING):
        pltpu.make_async_copy(
            e_hbm.at[pl.ds(0, 128)],
            rows_v.at[pl.ds(i * 128, 128)], sem_s).wait()

      @pl.when(c + 1 < NCH)
      def _():
        pltpu.make_async_copy(src_hbm.at[pl.ds(0, CH)],
                              src_v.at[pl.ds(0, CH)], sem_i).wait()
        pltpu.make_async_copy(dst_hbm.at[pl.ds(0, CH)],
                              dst_v.at[pl.ds(0, CH)], sem_i).wait()
      return carry

    lax.fori_loop(0, NCH, chunk_body, None)
    plsc.subcore_barrier()

    pltpu.sync_copy(g_sh.at[pl.ds(sid * WROWS_PS, WROWS_PS)],
                    g_hbm.at[pl.ds(base + sid * WROWS_PS, WROWS_PS)])

  return k(e, src_2d, dst_2d)


TB = 512
TC_GRID = N_G // TB


def _tc_body(e_ref, g_ref, w_ref, m_ref, o_ref, acc_ref):
  b = pl.program_id(0)
  x = e_ref[...]
  s = lax.dot_general(x, w_ref[...], (((1,), (1,)), ((), ())),
                      preferred_element_type=jnp.float32)
  s += lax.dot_general(g_ref[...], m_ref[...], (((1,), (1,)), ((), ())),
                       preferred_element_type=jnp.float32)
  s = jnp.maximum(s, 0.0)
  row = b * TB + lax.broadcasted_iota(jnp.int32, (TB, 1), 0)
  s = jnp.where(row < N, s, 0.0)

  @pl.when(b == 0)
  def _():
    acc_ref[...] = jnp.zeros((8, D), jnp.float32)

  acc_ref[...] += jnp.sum(s.reshape(TB // 8, 8, D), axis=0)

  @pl.when(b == TC_GRID - 1)
  def _():
    r = jnp.sum(acc_ref[...], axis=0, keepdims=True)
    mx = jnp.max(r)
    ex = jnp.exp(r - mx)
    o_ref[...] = ex / jnp.sum(ex)


def _tc_reduce(e, g, W, M):
  return pl.pallas_call(
      _tc_body,
      grid=(TC_GRID,),
      in_specs=[
          pl.BlockSpec((TB, D), lambda b: (b, 0)),
          pl.BlockSpec((TB, D), lambda b: (b, 0)),
          pl.BlockSpec((D, D), lambda b: (0, 0)),
          pl.BlockSpec((D, D), lambda b: (0, 0)),
      ],
      out_specs=pl.BlockSpec((1, D), lambda b: (0, 0)),
      out_shape=jax.ShapeDtypeStruct((1, D), jnp.float32),
      scratch_shapes=[pltpu.VMEM((8, D), jnp.float32)],
  )(e, g, W, M)


@jax.jit
def kernel(node_ids, edge_index, W, M, emb):
  nids = jnp.concatenate(
      [node_ids, jnp.zeros((N_PAD - N,), jnp.int32)]).astype(jnp.int32)
  src = jnp.concatenate(
      [edge_index[0], jnp.zeros((E_PAD - E,), jnp.int32)]).astype(jnp.int32)
  dst = jnp.concatenate(
      [edge_index[1], jnp.full((E_PAD - E,), N, jnp.int32)]).astype(jnp.int32)

  e = _sc_gather_e(emb, nids)
  g = _sc_edge_agg(e, src.reshape(E_PAD // 128, 128),
                   dst.reshape(E_PAD // 128, 128))
  out = _tc_reduce(e, g, W, M)
  return out.reshape(D, 1)
